# initial kernel scaffold (unmeasured)
import functools

import jax
import jax.numpy as jnp
from jax import lax
from jax.experimental import pallas as pl
from jax.experimental.pallas import tpu as pltpu

N_DEV = 16


def kernel(x, w_mat, scale_x, scale_w):
    m, k_per = x.shape
    k, n = w_mat.shape
    m_per = m // N_DEV

    def body(x_ref, w_ref, sx_ref, sw_ref, out_ref,
             xsend_ref, xg_ref, send_sems, recv_sems):
        my = lax.axis_index("i")

        for j in range(N_DEV):
            xsend_ref[j, :, :] = x_ref[j * m_per:(j + 1) * m_per, :].astype(
                jnp.float8_e4m3fn)

        barrier = pltpu.get_barrier_semaphore()
        for off in range(1, N_DEV):
            peer = lax.rem(my + off, N_DEV)
            pl.semaphore_signal(barrier, inc=1, device_id=(peer,),
                                device_id_type=pl.DeviceIdType.MESH)
        pl.semaphore_wait(barrier, N_DEV - 1)

        sends = []
        for off in range(1, N_DEV):
            dst = lax.rem(my + off, N_DEV)
            rdma = pltpu.make_async_remote_copy(
                src_ref=xsend_ref.at[dst],
                dst_ref=xg_ref.at[my],
                send_sem=send_sems.at[off],
                recv_sem=recv_sems.at[off],
                device_id=(dst,),
                device_id_type=pl.DeviceIdType.MESH,
            )
            rdma.start()
            sends.append(rdma)

        xg_ref[my, :, :] = xsend_ref[my, :, :]

        for off in range(1, N_DEV):
            src = lax.rem(my - off + N_DEV, N_DEV)
            recv = pltpu.make_async_remote_copy(
                src_ref=xg_ref.at[src],
                dst_ref=xg_ref.at[src],
                send_sem=send_sems.at[off],
                recv_sem=recv_sems.at[off],
                device_id=(src,),
                device_id_type=pl.DeviceIdType.MESH,
            )
            recv.wait_recv()

        acc = None
        for j in range(N_DEV):
            wblk = w_ref[j * k_per:(j + 1) * k_per, :].astype(jnp.float8_e5m2)
            p = lax.dot_general(
                xg_ref[j, :, :], wblk,
                (((1,), (0,)), ((), ())),
                preferred_element_type=jnp.float32,
            )
            acc = p if acc is None else acc + p

        out_ref[:, :] = acc * (sx_ref[0] * sw_ref[0])

        for rdma in sends:
            rdma.wait_send()

        @functools.partial(pl.run_scoped,
                           second_barrier=pltpu.SemaphoreType.REGULAR)
        def _(second_barrier):
            for off in range(1, N_DEV):
                peer = lax.rem(my + off, N_DEV)
                pl.semaphore_signal(second_barrier, inc=1, device_id=(peer,),
                                    device_id_type=pl.DeviceIdType.MESH)
            pl.semaphore_wait(second_barrier, N_DEV - 1)

    return pl.pallas_call(
        body,
        out_shape=jax.ShapeDtypeStruct((m_per, n), jnp.float32),
        in_specs=[
            pl.BlockSpec(memory_space=pltpu.VMEM),
            pl.BlockSpec(memory_space=pltpu.VMEM),
            pl.BlockSpec(memory_space=pltpu.SMEM),
            pl.BlockSpec(memory_space=pltpu.SMEM),
        ],
        out_specs=pl.BlockSpec(memory_space=pltpu.VMEM),
        scratch_shapes=[
            pltpu.VMEM((N_DEV, m_per, k_per), jnp.float8_e4m3fn),
            pltpu.VMEM((N_DEV, m_per, k_per), jnp.float8_e4m3fn),
            pltpu.SemaphoreType.DMA((N_DEV,)),
            pltpu.SemaphoreType.DMA((N_DEV,)),
        ],
        compiler_params=pltpu.CompilerParams(collective_id=0),
    )(x, w_mat, scale_x, scale_w)


# baseline (device time: 42024 ns/iter reference)
import functools

import jax
import jax.numpy as jnp
from jax import lax
from jax.experimental import pallas as pl
from jax.experimental.pallas import tpu as pltpu

N_DEV = 16


def kernel(x, w_mat, scale_x, scale_w):
    m, k_per = x.shape
    k, n = w_mat.shape
    m_per = m // N_DEV

    def body(x_ref, w_ref, sx_ref, sw_ref, out_ref,
             xsend_ref, xg_ref, send_sems, recv_sems):
        my = lax.axis_index("i")

        for j in range(N_DEV):
            xsend_ref[j, :, :] = x_ref[j * m_per:(j + 1) * m_per, :].astype(
                jnp.float8_e4m3fn)

        barrier = pltpu.get_barrier_semaphore()
        for off in range(1, N_DEV):
            peer = lax.rem(my + off, N_DEV)
            pl.semaphore_signal(barrier, inc=1, device_id=(peer,),
                                device_id_type=pl.DeviceIdType.MESH)
        pl.semaphore_wait(barrier, N_DEV - 1)

        sends = []
        for off in range(1, N_DEV):
            dst = lax.rem(my + off, N_DEV)
            rdma = pltpu.make_async_remote_copy(
                src_ref=xsend_ref.at[dst],
                dst_ref=xg_ref.at[my],
                send_sem=send_sems.at[off],
                recv_sem=recv_sems.at[off],
                device_id=(dst,),
                device_id_type=pl.DeviceIdType.MESH,
            )
            rdma.start()
            sends.append(rdma)

        xg_ref[my, :, :] = xsend_ref[my, :, :]

        for off in range(1, N_DEV):
            src = lax.rem(my - off + N_DEV, N_DEV)
            recv = pltpu.make_async_remote_copy(
                src_ref=xg_ref.at[src],
                dst_ref=xg_ref.at[src],
                send_sem=send_sems.at[off],
                recv_sem=recv_sems.at[off],
                device_id=(src,),
                device_id_type=pl.DeviceIdType.MESH,
            )
            recv.wait_recv()

        acc = None
        for j in range(N_DEV):
            wblk = w_ref[j * k_per:(j + 1) * k_per, :].astype(jnp.float8_e5m2)
            p = lax.dot_general(
                xg_ref[j, :, :], wblk,
                (((1,), (0,)), ((), ())),
                preferred_element_type=jnp.float32,
            )
            acc = p if acc is None else acc + p

        out_ref[:, :] = acc * (sx_ref[0] * sw_ref[0])

        for rdma in sends:
            rdma.wait_send()

        @functools.partial(pl.run_scoped,
                           second_barrier=pltpu.SemaphoreType.REGULAR)
        def _(second_barrier):
            for off in range(1, N_DEV):
                peer = lax.rem(my + off, N_DEV)
                pl.semaphore_signal(second_barrier, inc=1, device_id=(peer,),
                                    device_id_type=pl.DeviceIdType.MESH)
            pl.semaphore_wait(second_barrier, N_DEV - 1)

    return pl.pallas_call(
        body,
        out_shape=jax.ShapeDtypeStruct((m_per, n), jnp.float32),
        in_specs=[
            pl.BlockSpec(memory_space=pltpu.VMEM),
            pl.BlockSpec(memory_space=pltpu.VMEM),
            pl.BlockSpec(memory_space=pltpu.SMEM),
            pl.BlockSpec(memory_space=pltpu.SMEM),
        ],
        out_specs=pl.BlockSpec(memory_space=pltpu.VMEM),
        scratch_shapes=[
            pltpu.VMEM((N_DEV, m_per, k_per), jnp.float8_e4m3fn),
            pltpu.VMEM((N_DEV, m_per, k_per), jnp.float8_e4m3fn),
            pltpu.SemaphoreType.DMA((N_DEV,)),
            pltpu.SemaphoreType.DMA((N_DEV,)),
        ],
        compiler_params=pltpu.CompilerParams(
            collective_id=0, vmem_limit_bytes=100 * 1024 * 1024),
    )(x, w_mat, scale_x, scale_w)


# device time: 32862 ns/iter; 1.2788x vs baseline; 1.2788x over previous
import functools

import jax
import jax.numpy as jnp
from jax import lax
from jax.experimental import pallas as pl
from jax.experimental.pallas import tpu as pltpu

N_DEV = 16
NBUF = 4


def kernel(x, w_mat, scale_x, scale_w):
    m, k_per = x.shape
    k, n = w_mat.shape
    m_per = m // N_DEV

    def body(x_ref, w_hbm, sx_ref, sw_ref, out_ref,
             xsend_ref, xg_ref, wbuf_ref, send_sems, recv_sems, wdma_sems):
        my = lax.axis_index("i")

        def wsrc(t):
            return lax.rem(my - t + N_DEV, N_DEV)

        def wdma(t):
            return pltpu.make_async_copy(
                w_hbm.at[pl.ds(wsrc(t) * k_per, k_per), :],
                wbuf_ref.at[t % NBUF],
                wdma_sems.at[t % NBUF],
            )

        for t in range(NBUF):
            wdma(t).start()

        for j in range(N_DEV):
            xsend_ref[j, :, :] = x_ref[j * m_per:(j + 1) * m_per, :].astype(
                jnp.float8_e4m3fn)

        barrier = pltpu.get_barrier_semaphore()
        for off in range(1, N_DEV):
            peer = lax.rem(my + off, N_DEV)
            pl.semaphore_signal(barrier, inc=1, device_id=(peer,),
                                device_id_type=pl.DeviceIdType.MESH)
        pl.semaphore_wait(barrier, N_DEV - 1)

        sends = []
        for off in range(1, N_DEV):
            dst = lax.rem(my + off, N_DEV)
            rdma = pltpu.make_async_remote_copy(
                src_ref=xsend_ref.at[dst],
                dst_ref=xg_ref.at[my],
                send_sem=send_sems.at[off],
                recv_sem=recv_sems.at[off],
                device_id=(dst,),
                device_id_type=pl.DeviceIdType.MESH,
            )
            rdma.start()
            sends.append(rdma)

        xg_ref[my, :, :] = xsend_ref[my, :, :]

        acc = None
        for t in range(N_DEV):
            if t > 0:
                src = wsrc(t)
                recv = pltpu.make_async_remote_copy(
                    src_ref=xg_ref.at[src],
                    dst_ref=xg_ref.at[src],
                    send_sem=send_sems.at[t],
                    recv_sem=recv_sems.at[t],
                    device_id=(src,),
                    device_id_type=pl.DeviceIdType.MESH,
                )
                recv.wait_recv()
            wdma(t).wait()
            wblk = wbuf_ref[t % NBUF].astype(jnp.float8_e5m2)
            if t + NBUF < N_DEV:
                wdma(t + NBUF).start()
            p = lax.dot_general(
                xg_ref[wsrc(t)], wblk,
                (((1,), (0,)), ((), ())),
                preferred_element_type=jnp.float32,
            )
            acc = p if acc is None else acc + p

        out_ref[:, :] = acc * (sx_ref[0] * sw_ref[0])

        for rdma in sends:
            rdma.wait_send()

        @functools.partial(pl.run_scoped,
                           second_barrier=pltpu.SemaphoreType.REGULAR)
        def _(second_barrier):
            for off in range(1, N_DEV):
                peer = lax.rem(my + off, N_DEV)
                pl.semaphore_signal(second_barrier, inc=1, device_id=(peer,),
                                    device_id_type=pl.DeviceIdType.MESH)
            pl.semaphore_wait(second_barrier, N_DEV - 1)

    return pl.pallas_call(
        body,
        out_shape=jax.ShapeDtypeStruct((m_per, n), jnp.float32),
        in_specs=[
            pl.BlockSpec(memory_space=pltpu.VMEM),
            pl.BlockSpec(memory_space=pl.ANY),
            pl.BlockSpec(memory_space=pltpu.SMEM),
            pl.BlockSpec(memory_space=pltpu.SMEM),
        ],
        out_specs=pl.BlockSpec(memory_space=pltpu.VMEM),
        scratch_shapes=[
            pltpu.VMEM((N_DEV, m_per, k_per), jnp.float8_e4m3fn),
            pltpu.VMEM((N_DEV, m_per, k_per), jnp.float8_e4m3fn),
            pltpu.VMEM((NBUF, k_per, n), jnp.float32),
            pltpu.SemaphoreType.DMA((N_DEV,)),
            pltpu.SemaphoreType.DMA((N_DEV,)),
            pltpu.SemaphoreType.DMA((NBUF,)),
        ],
        compiler_params=pltpu.CompilerParams(
            collective_id=0, vmem_limit_bytes=100 * 1024 * 1024),
    )(x, w_mat, scale_x, scale_w)


# device time: 26815 ns/iter; 1.5672x vs baseline; 1.2255x over previous
import os

import jax
import jax.numpy as jnp
from jax import lax
from jax.experimental import pallas as pl
from jax.experimental.pallas import tpu as pltpu

N_DEV = 16

_BY_DIST = sorted(range(1, N_DEV), key=lambda o: (min(o, N_DEV - o), o))
PROC_ORDER = [0] + _BY_DIST
SEND_ORDER = list(reversed(_BY_DIST))

SEND_WINDOW = int(os.environ.get("KERNEL_SEND_WINDOW", "16"))

_VARIANT = os.environ.get("KERNEL_VARIANT", "full")
_COMM = _VARIANT not in ("nocomm",)
_GEMM = _VARIANT not in ("nogemm", "nosend", "sendnorecv")
_SEND = _COMM and _VARIANT not in ("nosend",)
_RECV = _SEND and _VARIANT not in ("sendnorecv",)


def kernel(x, w_mat, scale_x, scale_w):
    m, k_per = x.shape
    k, n = w_mat.shape
    m_per = m // N_DEV

    def body(x_ref, w_hbm, sx_ref, sw_ref, out_ref,
             xsend_ref, xg_ref, wbuf_ref, we5_ref, send_sems, recv_sems,
             wdma_sems, ready_sems):
        my = lax.axis_index("i")

        def wsrc(off):
            return lax.rem(my - off + N_DEV, N_DEV)

        def wdma(t):
            return pltpu.make_async_copy(
                w_hbm.at[pl.ds(wsrc(PROC_ORDER[t]) * k_per, k_per), :],
                wbuf_ref.at[t],
                wdma_sems.at[t],
            )

        if _COMM:
            for off in SEND_ORDER:
                pl.semaphore_signal(ready_sems.at[off], inc=1,
                                    device_id=(wsrc(off),),
                                    device_id_type=pl.DeviceIdType.MESH)

        if _GEMM:
            for t in range(N_DEV):
                wdma(t).start()

        sends = []
        if _COMM:
            barrier = pltpu.get_barrier_semaphore()
            for nbr in (lax.rem(my + 1, N_DEV), lax.rem(my - 1 + N_DEV, N_DEV)):
                pl.semaphore_signal(barrier, inc=1, device_id=(nbr,),
                                    device_id_type=pl.DeviceIdType.MESH)
            pl.semaphore_wait(barrier, 2)

            for idx, off in enumerate(SEND_ORDER):
                dst = lax.rem(my + off, N_DEV)
                xsend_ref[dst, :, :] = x_ref[
                    pl.ds(dst * m_per, m_per), :].astype(jnp.float8_e4m3fn)
                pl.semaphore_wait(ready_sems.at[off], 1)
                if _SEND:
                    if idx >= SEND_WINDOW:
                        sends[idx - SEND_WINDOW].wait_send()
                    rdma = pltpu.make_async_remote_copy(
                        src_ref=xsend_ref.at[dst],
                        dst_ref=xg_ref.at[my],
                        send_sem=send_sems.at[off],
                        recv_sem=recv_sems.at[off],
                        device_id=(dst,),
                        device_id_type=pl.DeviceIdType.MESH,
                    )
                    rdma.start()
                    sends.append(rdma)

            xg_ref[my, :, :] = x_ref[
                pl.ds(my * m_per, m_per), :].astype(jnp.float8_e4m3fn)
        else:
            for j in range(N_DEV):
                xsend_ref[j, :, :] = x_ref[
                    j * m_per:(j + 1) * m_per, :].astype(jnp.float8_e4m3fn)

        if _GEMM:
            for t in range(N_DEV):
                wdma(t).wait()
                we5_ref[t, :, :] = wbuf_ref[t, :, :].astype(jnp.float8_e5m2)

        acc = None
        for t in range(N_DEV):
            off = PROC_ORDER[t]
            if _RECV and t > 0:
                src = wsrc(off)
                recv = pltpu.make_async_remote_copy(
                    src_ref=xg_ref.at[src],
                    dst_ref=xg_ref.at[src],
                    send_sem=send_sems.at[off],
                    recv_sem=recv_sems.at[off],
                    device_id=(src,),
                    device_id_type=pl.DeviceIdType.MESH,
                )
                recv.wait_recv()
            if _GEMM:
                xsrc = xg_ref if _COMM else xsend_ref
                p = lax.dot_general(
                    xsrc[wsrc(off)], we5_ref[t, :, :],
                    (((1,), (0,)), ((), ())),
                    preferred_element_type=jnp.float32,
                )
                acc = p if acc is None else acc + p

        if _GEMM:
            out_ref[:, :] = acc * (sx_ref[0] * sw_ref[0])
        else:
            out_ref[:, :] = jnp.zeros((m_per, n), jnp.float32)

        for rdma in sends[-SEND_WINDOW:]:
            rdma.wait_send()

    return pl.pallas_call(
        body,
        out_shape=jax.ShapeDtypeStruct((m_per, n), jnp.float32),
        in_specs=[
            pl.BlockSpec(memory_space=pltpu.VMEM),
            pl.BlockSpec(memory_space=pl.ANY),
            pl.BlockSpec(memory_space=pltpu.SMEM),
            pl.BlockSpec(memory_space=pltpu.SMEM),
        ],
        out_specs=pl.BlockSpec(memory_space=pltpu.VMEM),
        scratch_shapes=[
            pltpu.VMEM((N_DEV, m_per, k_per), jnp.float8_e4m3fn),
            pltpu.VMEM((N_DEV, m_per, k_per), jnp.float8_e4m3fn),
            pltpu.VMEM((N_DEV, k_per, n), jnp.float32),
            pltpu.VMEM((N_DEV, k_per, n), jnp.float8_e5m2),
            pltpu.SemaphoreType.DMA((N_DEV,)),
            pltpu.SemaphoreType.DMA((N_DEV,)),
            pltpu.SemaphoreType.DMA((N_DEV,)),
            pltpu.SemaphoreType.REGULAR((N_DEV,)),
        ],
        compiler_params=pltpu.CompilerParams(
            collective_id=0 if _COMM else None,
            vmem_limit_bytes=100 * 1024 * 1024),
    )(x, w_mat, scale_x, scale_w)


# device time: 24722 ns/iter; 1.6999x vs baseline; 1.0847x over previous
import os

import jax
import jax.numpy as jnp
from jax import lax
from jax.experimental import pallas as pl
from jax.experimental.pallas import tpu as pltpu

N_DEV = 16

_BY_DIST = sorted(range(1, N_DEV), key=lambda o: (min(o, N_DEV - o), o))
PROC_ORDER = [0] + _BY_DIST
SEND_ORDER = (list(reversed(_BY_DIST))
              if os.environ.get("KERNEL_SEND_FAR") == "1"
              else list(_BY_DIST))

SEND_WINDOW = int(os.environ.get("KERNEL_SEND_WINDOW", "16"))

_VARIANT = os.environ.get("KERNEL_VARIANT", "full")
_COMM = _VARIANT not in ("nocomm",)
_GEMM = _VARIANT not in ("nogemm", "nosend", "sendnorecv")
_SEND = _COMM and _VARIANT not in ("nosend",)
_RECV = _SEND and _VARIANT not in ("sendnorecv",)


def kernel(x, w_mat, scale_x, scale_w):
    m, k_per = x.shape
    k, n = w_mat.shape
    m_per = m // N_DEV

    def body(x_ref, w_hbm, sx_ref, sw_ref, out_ref,
             xsend_ref, xg_ref, wbuf_ref, we5_ref, send_sems, recv_sems,
             wdma_sems, ready_sems):
        my = lax.axis_index("i")

        def wsrc(off):
            return lax.rem(my - off + N_DEV, N_DEV)

        def wdma(t):
            return pltpu.make_async_copy(
                w_hbm.at[pl.ds(wsrc(PROC_ORDER[t]) * k_per, k_per), :],
                wbuf_ref.at[t],
                wdma_sems.at[t],
            )

        if _COMM:
            for off in SEND_ORDER:
                pl.semaphore_signal(ready_sems.at[off], inc=1,
                                    device_id=(wsrc(off),),
                                    device_id_type=pl.DeviceIdType.MESH)

        if _GEMM:
            for t in range(N_DEV):
                wdma(t).start()

        sends = []
        if _COMM:
            barrier = pltpu.get_barrier_semaphore()
            for nbr in (lax.rem(my + 1, N_DEV), lax.rem(my - 1 + N_DEV, N_DEV)):
                pl.semaphore_signal(barrier, inc=1, device_id=(nbr,),
                                    device_id_type=pl.DeviceIdType.MESH)
            pl.semaphore_wait(barrier, 2)

            for idx, off in enumerate(SEND_ORDER):
                dst = lax.rem(my + off, N_DEV)
                xsend_ref[dst, :, :] = x_ref[
                    pl.ds(dst * m_per, m_per), :].astype(jnp.float8_e4m3fn)
                pl.semaphore_wait(ready_sems.at[off], 1)
                if _SEND:
                    if idx >= SEND_WINDOW:
                        sends[idx - SEND_WINDOW].wait_send()
                    rdma = pltpu.make_async_remote_copy(
                        src_ref=xsend_ref.at[dst],
                        dst_ref=xg_ref.at[my],
                        send_sem=send_sems.at[off],
                        recv_sem=recv_sems.at[off],
                        device_id=(dst,),
                        device_id_type=pl.DeviceIdType.MESH,
                    )
                    rdma.start()
                    sends.append(rdma)

            xg_ref[my, :, :] = x_ref[
                pl.ds(my * m_per, m_per), :].astype(jnp.float8_e4m3fn)
        else:
            for j in range(N_DEV):
                xsend_ref[j, :, :] = x_ref[
                    j * m_per:(j + 1) * m_per, :].astype(jnp.float8_e4m3fn)

        if _GEMM:
            for t in range(N_DEV):
                wdma(t).wait()
                we5_ref[t, :, :] = wbuf_ref[t, :, :].astype(jnp.float8_e5m2)

        acc = None
        for t in range(N_DEV):
            off = PROC_ORDER[t]
            if _RECV and t > 0:
                src = wsrc(off)
                recv = pltpu.make_async_remote_copy(
                    src_ref=xg_ref.at[src],
                    dst_ref=xg_ref.at[src],
                    send_sem=send_sems.at[off],
                    recv_sem=recv_sems.at[off],
                    device_id=(src,),
                    device_id_type=pl.DeviceIdType.MESH,
                )
                recv.wait_recv()
            if _GEMM:
                xsrc = xg_ref if _COMM else xsend_ref
                p = lax.dot_general(
                    xsrc[wsrc(off)], we5_ref[t, :, :],
                    (((1,), (0,)), ((), ())),
                    preferred_element_type=jnp.float32,
                )
                acc = p if acc is None else acc + p

        if _GEMM:
            out_ref[:, :] = acc * (sx_ref[0] * sw_ref[0])
        else:
            out_ref[:, :] = jnp.zeros((m_per, n), jnp.float32)

        for rdma in sends[-SEND_WINDOW:]:
            rdma.wait_send()

    return pl.pallas_call(
        body,
        out_shape=jax.ShapeDtypeStruct((m_per, n), jnp.float32),
        in_specs=[
            pl.BlockSpec(memory_space=pltpu.VMEM),
            pl.BlockSpec(memory_space=pl.ANY),
            pl.BlockSpec(memory_space=pltpu.SMEM),
            pl.BlockSpec(memory_space=pltpu.SMEM),
        ],
        out_specs=pl.BlockSpec(memory_space=pltpu.VMEM),
        scratch_shapes=[
            pltpu.VMEM((N_DEV, m_per, k_per), jnp.float8_e4m3fn),
            pltpu.VMEM((N_DEV, m_per, k_per), jnp.float8_e4m3fn),
            pltpu.VMEM((N_DEV, k_per, n), jnp.float32),
            pltpu.VMEM((N_DEV, k_per, n), jnp.float8_e5m2),
            pltpu.SemaphoreType.DMA((N_DEV,)),
            pltpu.SemaphoreType.DMA((N_DEV,)),
            pltpu.SemaphoreType.DMA((N_DEV,)),
            pltpu.SemaphoreType.REGULAR((N_DEV,)),
        ],
        compiler_params=pltpu.CompilerParams(
            collective_id=0 if _COMM else None,
            vmem_limit_bytes=100 * 1024 * 1024),
    )(x, w_mat, scale_x, scale_w)
